# baseline (speedup 1.0000x reference)
import functools

import jax
import jax.numpy as jnp
from jax import lax
from jax.experimental import pallas as pl
from jax.experimental.pallas import tpu as pltpu
from jax.experimental.pallas import tpu_sc as plsc

R, C = 128, 32768
K = 16
NC, NS, L = 2, 16, 16
NW = NC * NS
ROWS_PER_W = R // NW
CHUNKS = C // L


def _scan_row(buf, lane):
    x0 = buf[pl.ds(0, L)]
    x0 = jnp.where(x0 != x0, jnp.float32(1e10), x0)
    bv, bi = plsc.sort_key_val(x0, lane)
    thr = jnp.broadcast_to(jnp.max(bv), (L,))

    def step(j, carry):
        bv, bi, thr = carry
        x = buf[pl.ds(j * L, L)]
        x = jnp.where(x != x, jnp.float32(1e10), x)
        hit = jnp.any(x < thr)

        def upd(args):
            bv, bi, _ = args
            idxv = lane + j * L
            xs, xi = plsc.sort_key_val(x, idxv)
            rxs = lax.rev(xs, (0,))
            rxi = lax.rev(xi, (0,))
            take_b = bv <= rxs
            lo = jnp.where(take_b, bv, rxs)
            li = jnp.where(take_b, bi, rxi)
            nbv, nbi = plsc.sort_key_val(lo, li)
            nthr = jnp.broadcast_to(jnp.max(nbv), (L,))
            return nbv, nbi, nthr

        return lax.cond(hit, upd, lambda a: a, (bv, bi, thr))

    bv, bi, thr = lax.fori_loop(1, CHUNKS, step, (bv, bi, thr))
    return bv, bi


def _sc_body(x_hbm, idx_hbm, val_hbm, buf0, buf1, sti, stv, sem0, sem1):
    wid = lax.axis_index("s") * NC + lax.axis_index("c")
    row0 = wid * ROWS_PER_W
    lane = lax.iota(jnp.int32, 16)

    bufs = (buf0, buf1)
    sems = (sem0, sem1)
    cp = pltpu.async_copy(x_hbm.at[row0], buf0, sem0)
    for r in range(ROWS_PER_W):
        cp.wait()
        if r + 1 < ROWS_PER_W:
            nxt = pltpu.async_copy(
                x_hbm.at[row0 + (r + 1)], bufs[(r + 1) % 2], sems[(r + 1) % 2]
            )
        bv, bi = _scan_row(bufs[r % 2], lane)
        stv[r] = bv
        sti[r] = bi
        if r + 1 < ROWS_PER_W:
            cp = nxt

    pltpu.sync_copy(sti, idx_hbm.at[pl.ds(row0, ROWS_PER_W)])
    pltpu.sync_copy(stv, val_hbm.at[pl.ds(row0, ROWS_PER_W)])


@jax.jit
def _bottom_k(x):
    mesh = plsc.VectorSubcoreMesh(core_axis_name="c", subcore_axis_name="s")
    return pl.kernel(
        _sc_body,
        out_type=[
            jax.ShapeDtypeStruct((R, K), jnp.int32),
            jax.ShapeDtypeStruct((R, K), jnp.float32),
        ],
        mesh=mesh,
        compiler_params=pltpu.CompilerParams(needs_layout_passes=False),
        scratch_types=[
            pltpu.VMEM((C,), jnp.float32),
            pltpu.VMEM((C,), jnp.float32),
            pltpu.VMEM((ROWS_PER_W, K), jnp.int32),
            pltpu.VMEM((ROWS_PER_W, K), jnp.float32),
            pltpu.SemaphoreType.DMA,
            pltpu.SemaphoreType.DMA,
        ],
    )(x)


def kernel(dist_pot_donors, n_neighbors):
    idx, vals = _bottom_k(dist_pot_donors)
    idx = idx + (jnp.asarray(n_neighbors, dtype=idx.dtype) - K)
    return (idx, vals)


# speedup vs baseline: 3.0237x; 3.0237x over previous
import functools

import jax
import jax.numpy as jnp
from jax import lax
from jax.experimental import pallas as pl
from jax.experimental.pallas import tpu as pltpu
from jax.experimental.pallas import tpu_sc as plsc

R, C = 128, 32768
K = 16
NC, NS, L = 2, 16, 16
NW = NC * NS
ROWS_PER_W = R // NW
CHUNKS = C // L


U = 8
BIG = 1e10


def _merge(bv, bi, x, idxv):
    xs, xi = plsc.sort_key_val(x, idxv)
    rxs = lax.rev(xs, (0,))
    rxi = lax.rev(xi, (0,))
    take_b = bv <= rxs
    lo = jnp.where(take_b, bv, rxs)
    li = jnp.where(take_b, bi, rxi)
    return plsc.sort_key_val(lo, li)


def _scan_row(buf, lane):
    x0 = buf[pl.ds(0, L)]
    x0 = jnp.where(x0 != x0, BIG, x0)
    bv, bi = plsc.sort_key_val(x0, lane)
    for u in range(1, U):
        x = buf[pl.ds(u * L, L)]
        x = jnp.where(x != x, BIG, x)
        bv, bi = _merge(bv, bi, x, lane + u * L)
    thr = jnp.max(bv)

    def step(j, carry):
        base = j * (U * L)
        xs = []
        for u in range(U):
            x = buf[pl.ds(base + u * L, L)]
            xs.append(jnp.where(x != x, BIG, x))
        m = xs[0]
        for u in range(1, U):
            m = jnp.minimum(m, xs[u])
        hit = jnp.min(m) < carry[2]

        def upd(args):
            bv, bi, thr = args
            for u in range(U):

                def mrg(a, _x=xs[u], _iv=lane + u * L):
                    nbv, nbi = _merge(a[0], a[1], _x, _iv + base)
                    return nbv, nbi, jnp.max(nbv)

                bv, bi, thr = lax.cond(
                    jnp.min(xs[u]) < thr, mrg, lambda a: a, (bv, bi, thr)
                )
            return bv, bi, thr

        return lax.cond(hit, upd, lambda a: a, carry)

    bv, bi, thr = lax.fori_loop(1, CHUNKS // U, step, (bv, bi, thr))
    return bv, bi


def _sc_body(x_hbm, idx_hbm, val_hbm, buf0, buf1, sti, stv, sem0, sem1):
    wid = lax.axis_index("s") * NC + lax.axis_index("c")
    row0 = wid * ROWS_PER_W
    lane = lax.iota(jnp.int32, 16)

    bufs = (buf0, buf1)
    sems = (sem0, sem1)
    cp = pltpu.async_copy(x_hbm.at[row0], buf0, sem0)
    for r in range(ROWS_PER_W):
        cp.wait()
        if r + 1 < ROWS_PER_W:
            nxt = pltpu.async_copy(
                x_hbm.at[row0 + (r + 1)], bufs[(r + 1) % 2], sems[(r + 1) % 2]
            )
        bv, bi = _scan_row(bufs[r % 2], lane)
        stv[r] = bv
        sti[r] = bi
        if r + 1 < ROWS_PER_W:
            cp = nxt

    pltpu.sync_copy(sti, idx_hbm.at[pl.ds(row0, ROWS_PER_W)])
    pltpu.sync_copy(stv, val_hbm.at[pl.ds(row0, ROWS_PER_W)])


@jax.jit
def _bottom_k(x):
    mesh = plsc.VectorSubcoreMesh(core_axis_name="c", subcore_axis_name="s")
    return pl.kernel(
        _sc_body,
        out_type=[
            jax.ShapeDtypeStruct((R, K), jnp.int32),
            jax.ShapeDtypeStruct((R, K), jnp.float32),
        ],
        mesh=mesh,
        compiler_params=pltpu.CompilerParams(needs_layout_passes=False),
        scratch_types=[
            pltpu.VMEM((C,), jnp.float32),
            pltpu.VMEM((C,), jnp.float32),
            pltpu.VMEM((ROWS_PER_W, K), jnp.int32),
            pltpu.VMEM((ROWS_PER_W, K), jnp.float32),
            pltpu.SemaphoreType.DMA,
            pltpu.SemaphoreType.DMA,
        ],
    )(x)


def kernel(dist_pot_donors, n_neighbors):
    idx, vals = _bottom_k(dist_pot_donors)
    idx = idx + (jnp.asarray(n_neighbors, dtype=idx.dtype) - K)
    return (idx, vals)


# speedup vs baseline: 3.1896x; 1.0549x over previous
import jax
import jax.numpy as jnp
from jax import lax
from jax.experimental import pallas as pl
from jax.experimental.pallas import tpu as pltpu
from jax.experimental.pallas import tpu_sc as plsc

R, C = 128, 32768
K = 16
NC, NS, L = 2, 16, 16
NW = NC * NS
ROWS_PER_W = R // NW
CHUNKS = C // L
U = 8
BIG = 1e10


def _any_below(x, thr):
    pc = plsc.all_reduce_population_count(x < thr)
    return pc[0] > 0


def _merge(bv, bi, x, idxv):
    x = jnp.where(x != x, BIG, x)
    xs, xi = plsc.sort_key_val(x, idxv)
    rxs = lax.rev(xs, (0,))
    rxi = lax.rev(xi, (0,))
    take_b = bv <= rxs
    lo = jnp.where(take_b, bv, rxs)
    li = jnp.where(take_b, bi, rxi)
    return plsc.sort_key_val(lo, li)


def _row_init(buf, lane):
    x0 = buf[pl.ds(0, L)]
    x0 = jnp.where(x0 != x0, BIG, x0)
    bv, bi = plsc.sort_key_val(x0, lane)
    for u in range(1, U):
        bv, bi = _merge(bv, bi, buf[pl.ds(u * L, L)], lane + u * L)
    return bv, bi, jnp.broadcast_to(bv[K - 1], (L,))


def _row_step(buf, lane, j, bv, bi, thr):
    base = j * (U * L)
    xs = [buf[pl.ds(base + u * L, L)] for u in range(U)]
    m = xs[0]
    for u in range(1, U):
        m = jnp.minimum(m, xs[u])

    def upd(args):
        bv, bi, thr = args
        for u in range(U):

            def mrg(a, _x=xs[u], _iv=lane + u * L):
                nbv, nbi = _merge(a[0], a[1], _x, _iv + base)
                return nbv, nbi, jnp.broadcast_to(nbv[K - 1], (L,))

            bv, bi, thr = lax.cond(
                _any_below(xs[u], thr), mrg, lambda a: a, (bv, bi, thr)
            )
        return bv, bi, thr

    return lax.cond(_any_below(m, thr), upd, lambda a: a, (bv, bi, thr))


def _scan_pair(bufA, bufB, lane):
    bvA, biA, thrA = _row_init(bufA, lane)
    bvB, biB, thrB = _row_init(bufB, lane)

    def step(j, carry):
        bvA, biA, thrA, bvB, biB, thrB = carry
        bvA, biA, thrA = _row_step(bufA, lane, j, bvA, biA, thrA)
        bvB, biB, thrB = _row_step(bufB, lane, j, bvB, biB, thrB)
        return bvA, biA, thrA, bvB, biB, thrB

    out = lax.fori_loop(1, CHUNKS // U, step, (bvA, biA, thrA, bvB, biB, thrB))
    return out[0], out[1], out[3], out[4]


def _sc_body(x_hbm, idx_hbm, val_hbm, buf0, buf1, buf2, sti, stv, s0, s1, s2):
    wid = lax.axis_index("s") * NC + lax.axis_index("c")
    row0 = wid * ROWS_PER_W
    lane = lax.iota(jnp.int32, 16)

    cpA = pltpu.async_copy(x_hbm.at[row0], buf0, s0)
    cpB = pltpu.async_copy(x_hbm.at[row0 + 1], buf1, s1)
    cpC = pltpu.async_copy(x_hbm.at[row0 + 2], buf2, s2)
    cpA.wait()
    cpB.wait()
    bv0, bi0, bv1, bi1 = _scan_pair(buf0, buf1, lane)
    stv[0] = bv0
    sti[0] = bi0
    stv[1] = bv1
    sti[1] = bi1

    cpD = pltpu.async_copy(x_hbm.at[row0 + 3], buf1, s1)
    cpC.wait()
    cpD.wait()
    bv2, bi2, bv3, bi3 = _scan_pair(buf2, buf1, lane)
    stv[2] = bv2
    sti[2] = bi2
    stv[3] = bv3
    sti[3] = bi3

    pltpu.sync_copy(sti, idx_hbm.at[pl.ds(row0, ROWS_PER_W)])
    pltpu.sync_copy(stv, val_hbm.at[pl.ds(row0, ROWS_PER_W)])


@jax.jit
def _bottom_k(x):
    mesh = plsc.VectorSubcoreMesh(core_axis_name="c", subcore_axis_name="s")
    return pl.kernel(
        _sc_body,
        out_type=[
            jax.ShapeDtypeStruct((R, K), jnp.int32),
            jax.ShapeDtypeStruct((R, K), jnp.float32),
        ],
        mesh=mesh,
        compiler_params=pltpu.CompilerParams(needs_layout_passes=False),
        scratch_types=[
            pltpu.VMEM((C,), jnp.float32),
            pltpu.VMEM((C,), jnp.float32),
            pltpu.VMEM((C,), jnp.float32),
            pltpu.VMEM((ROWS_PER_W, K), jnp.int32),
            pltpu.VMEM((ROWS_PER_W, K), jnp.float32),
            pltpu.SemaphoreType.DMA,
            pltpu.SemaphoreType.DMA,
            pltpu.SemaphoreType.DMA,
        ],
    )(x)


def kernel(dist_pot_donors, n_neighbors):
    idx, vals = _bottom_k(dist_pot_donors)
    idx = idx + (jnp.asarray(n_neighbors, dtype=idx.dtype) - K)
    return (idx, vals)


# speedup vs baseline: 3.3860x; 1.0616x over previous
import jax
import jax.numpy as jnp
from jax import lax
from jax.experimental import pallas as pl
from jax.experimental.pallas import tpu as pltpu
from jax.experimental.pallas import tpu_sc as plsc

R, C = 128, 32768
K = 16
NC, NS, L = 2, 16, 16
NW = NC * NS
ROWS_PER_W = R // NW
CHUNKS = C // L
U = 8
BIG = 1e10


def _any_below(x, thr):
    pc = plsc.all_reduce_population_count(x < thr)
    return pc[0] > 0


def _merge(bv, bi, x, idxv):
    x = jnp.where(x != x, BIG, x)
    xs, xi = plsc.sort_key_val(x, idxv)
    rxs = lax.rev(xs, (0,))
    rxi = lax.rev(xi, (0,))
    take_b = bv <= rxs
    lo = jnp.where(take_b, bv, rxs)
    li = jnp.where(take_b, bi, rxi)
    return plsc.sort_key_val(lo, li)


def _row_init(buf, lane):
    x0 = buf[pl.ds(0, L)]
    x0 = jnp.where(x0 != x0, BIG, x0)
    bv, bi = plsc.sort_key_val(x0, lane)
    for u in range(1, U):
        bv, bi = _merge(bv, bi, buf[pl.ds(u * L, L)], lane + u * L)
    return bv, bi, jnp.broadcast_to(bv[K - 1], (L,))


def _group_min(buf, base):
    xs = [buf[pl.ds(base + u * L, L)] for u in range(U)]
    while len(xs) > 1:
        xs = [jnp.minimum(xs[i], xs[i + 1]) for i in range(0, len(xs), 2)]
    return xs[0]


def _row_update(buf, lane, base, bv, bi, thr):
    for u in range(U):
        x = buf[pl.ds(base + u * L, L)]

        def mrg(a, _x=x, _iv=lane + u * L):
            nbv, nbi = _merge(a[0], a[1], _x, _iv + base)
            return nbv, nbi, jnp.broadcast_to(nbv[K - 1], (L,))

        bv, bi, thr = lax.cond(
            _any_below(x, thr), mrg, lambda a: a, (bv, bi, thr)
        )
    return bv, bi, thr


def _scan_pair(bufA, bufB, lane):
    bvA, biA, thrA = _row_init(bufA, lane)
    bvB, biB, thrB = _row_init(bufB, lane)
    GL = U * L
    mA0 = _group_min(bufA, GL)
    mB0 = _group_min(bufB, GL)

    def check_prev(j, state, mA_p, mB_p):
        bvA, biA, thrA, bvB, biB, thrB = state
        pbase = (j - 1) * GL

        def upd(args):
            bvA, biA, thrA, bvB, biB, thrB = args

            def updA(a):
                a2 = _row_update(bufA, lane, pbase, a[0], a[1], a[2])
                return a2 + a[3:]

            def updB(a):
                a2 = _row_update(bufB, lane, pbase, a[3], a[4], a[5])
                return a[:3] + a2

            args = lax.cond(_any_below(mA_p, args[2]), updA, lambda a: a, args)
            args = lax.cond(_any_below(mB_p, args[5]), updB, lambda a: a, args)
            return args

        hit = plsc.all_reduce_population_count(
            (mA_p < thrA) | (mB_p < thrB)
        )[0] > 0
        return lax.cond(hit, upd, lambda a: a, (bvA, biA, thrA, bvB, biB, thrB))

    def step(j, carry):
        mA_p, mB_p = carry[6], carry[7]
        mA = _group_min(bufA, j * GL)
        mB = _group_min(bufB, j * GL)
        state = check_prev(j, carry[:6], mA_p, mB_p)
        return state + (mA, mB)

    carry = (bvA, biA, thrA, bvB, biB, thrB, mA0, mB0)
    carry = lax.fori_loop(2, CHUNKS // U, step, carry)
    out = check_prev(CHUNKS // U, carry[:6], carry[6], carry[7])
    return out[0], out[1], out[3], out[4]


def _sc_body(x_hbm, idx_hbm, val_hbm, buf0, buf1, buf2, sti, stv, s0, s1, s2):
    wid = lax.axis_index("s") * NC + lax.axis_index("c")
    row0 = wid * ROWS_PER_W
    lane = lax.iota(jnp.int32, 16)

    cpA = pltpu.async_copy(x_hbm.at[row0], buf0, s0)
    cpB = pltpu.async_copy(x_hbm.at[row0 + 1], buf1, s1)
    cpC = pltpu.async_copy(x_hbm.at[row0 + 2], buf2, s2)
    cpA.wait()
    cpB.wait()
    bv0, bi0, bv1, bi1 = _scan_pair(buf0, buf1, lane)
    stv[0] = bv0
    sti[0] = bi0
    stv[1] = bv1
    sti[1] = bi1

    cpD = pltpu.async_copy(x_hbm.at[row0 + 3], buf1, s1)
    cpC.wait()
    cpD.wait()
    bv2, bi2, bv3, bi3 = _scan_pair(buf2, buf1, lane)
    stv[2] = bv2
    sti[2] = bi2
    stv[3] = bv3
    sti[3] = bi3

    pltpu.sync_copy(sti, idx_hbm.at[pl.ds(row0, ROWS_PER_W)])
    pltpu.sync_copy(stv, val_hbm.at[pl.ds(row0, ROWS_PER_W)])


@jax.jit
def _bottom_k(x):
    mesh = plsc.VectorSubcoreMesh(core_axis_name="c", subcore_axis_name="s")
    return pl.kernel(
        _sc_body,
        out_type=[
            jax.ShapeDtypeStruct((R, K), jnp.int32),
            jax.ShapeDtypeStruct((R, K), jnp.float32),
        ],
        mesh=mesh,
        compiler_params=pltpu.CompilerParams(needs_layout_passes=False),
        scratch_types=[
            pltpu.VMEM((C,), jnp.float32),
            pltpu.VMEM((C,), jnp.float32),
            pltpu.VMEM((C,), jnp.float32),
            pltpu.VMEM((ROWS_PER_W, K), jnp.int32),
            pltpu.VMEM((ROWS_PER_W, K), jnp.float32),
            pltpu.SemaphoreType.DMA,
            pltpu.SemaphoreType.DMA,
            pltpu.SemaphoreType.DMA,
        ],
    )(x)


def kernel(dist_pot_donors, n_neighbors):
    idx, vals = _bottom_k(dist_pot_donors)
    idx = idx + (jnp.asarray(n_neighbors, dtype=idx.dtype) - K)
    return (idx, vals)


# speedup vs baseline: 5.3921x; 1.5925x over previous
import jax
import jax.numpy as jnp
from jax import lax
from jax.experimental import pallas as pl
from jax.experimental.pallas import tpu as pltpu
from jax.experimental.pallas import tpu_sc as plsc

R, C = 128, 32768
K = 16
NC, NS, L = 2, 16, 16
NW = NC * NS
ROWS_PER_W = R // NW
CHUNKS = C // L
U = 8
GL = U * L
NG = CHUNKS // U
BIG = 1e10


def _any_below(x, thr):
    pc = plsc.all_reduce_population_count(x < thr)
    return pc[0] > 0


def _group_min(buf, base):
    xs = [buf[pl.ds(base + u * L, L)] for u in range(U)]
    while len(xs) > 1:
        xs = [jnp.minimum(xs[i], xs[i + 1]) for i in range(0, len(xs), 2)]
    return xs[0]


def _merge_sorted(av, ai, bv, bi):
    rbv = lax.rev(bv, (0,))
    rbi = lax.rev(bi, (0,))
    take_a = av <= rbv
    lo = jnp.where(take_a, av, rbv)
    li = jnp.where(take_a, ai, rbi)
    return plsc.sort_key_val(lo, li)


def _group_tree(buf, base, lane):
    pairs = []
    for u in range(U):
        x = buf[pl.ds(base + u * L, L)]
        x = jnp.where(x != x, BIG, x)
        pairs.append(plsc.sort_key_val(x, lane + base + u * L))
    while len(pairs) > 1:
        pairs = [
            _merge_sorted(*pairs[i], *pairs[i + 1])
            for i in range(0, len(pairs), 2)
        ]
    return pairs[0]


def _scan_pair(bufA, bufB, slotA, slotB, sti, stv, lane):
    gvA, giA = _group_tree(bufA, 0, lane)
    stv[slotA] = gvA
    sti[slotA] = giA
    gvB, giB = _group_tree(bufB, 0, lane)
    stv[slotB] = gvB
    sti[slotB] = giB
    thrA0 = jnp.broadcast_to(gvA[K - 1], (L,))
    thrB0 = jnp.broadcast_to(gvB[K - 1], (L,))

    def row_check(buf, slot, j, thr):
        def upd(_):
            gv, gi = _group_tree(buf, j * GL, lane)
            nbv, nbi = _merge_sorted(stv[slot], sti[slot], gv, gi)
            stv[slot] = nbv
            sti[slot] = nbi
            return jnp.broadcast_to(nbv[K - 1], (L,))

        m = _group_min(buf, j * GL)
        return lax.cond(_any_below(m, thr), upd, lambda _: thr, 0)

    def step(j, carry):
        thrA, thrB = carry
        thrA = row_check(bufA, slotA, j, thrA)
        thrB = row_check(bufB, slotB, j, thrB)
        return thrA, thrB

    lax.fori_loop(1, NG, step, (thrA0, thrB0))


def _sc_body(x_hbm, idx_hbm, val_hbm, buf0, buf1, buf2, sti, stv, s0, s1, s2):
    wid = lax.axis_index("s") * NC + lax.axis_index("c")
    row0 = wid * ROWS_PER_W
    lane = lax.iota(jnp.int32, 16)

    cpA = pltpu.async_copy(x_hbm.at[row0], buf0, s0)
    cpB = pltpu.async_copy(x_hbm.at[row0 + 1], buf1, s1)
    cpC = pltpu.async_copy(x_hbm.at[row0 + 2], buf2, s2)
    cpA.wait()
    cpB.wait()
    _scan_pair(buf0, buf1, 0, 1, sti, stv, lane)

    cpD = pltpu.async_copy(x_hbm.at[row0 + 3], buf1, s1)
    cpC.wait()
    cpD.wait()
    _scan_pair(buf2, buf1, 2, 3, sti, stv, lane)

    pltpu.sync_copy(sti, idx_hbm.at[pl.ds(row0, ROWS_PER_W)])
    pltpu.sync_copy(stv, val_hbm.at[pl.ds(row0, ROWS_PER_W)])


@jax.jit
def _bottom_k(x):
    mesh = plsc.VectorSubcoreMesh(core_axis_name="c", subcore_axis_name="s")
    return pl.kernel(
        _sc_body,
        out_type=[
            jax.ShapeDtypeStruct((R, K), jnp.int32),
            jax.ShapeDtypeStruct((R, K), jnp.float32),
        ],
        mesh=mesh,
        compiler_params=pltpu.CompilerParams(needs_layout_passes=False),
        scratch_types=[
            pltpu.VMEM((C,), jnp.float32),
            pltpu.VMEM((C,), jnp.float32),
            pltpu.VMEM((C,), jnp.float32),
            pltpu.VMEM((ROWS_PER_W, K), jnp.int32),
            pltpu.VMEM((ROWS_PER_W, K), jnp.float32),
            pltpu.SemaphoreType.DMA,
            pltpu.SemaphoreType.DMA,
            pltpu.SemaphoreType.DMA,
        ],
    )(x)


def kernel(dist_pot_donors, n_neighbors):
    idx, vals = _bottom_k(dist_pot_donors)
    idx = idx + (jnp.asarray(n_neighbors, dtype=idx.dtype) - K)
    return (idx, vals)
